# baseline (speedup 1.0000x reference)
import functools

import jax
import jax.numpy as jnp
from jax import lax
from jax.experimental import pallas as pl
from jax.experimental.pallas import tpu as pltpu
from jax.experimental.pallas import tpu_sc as plsc

_K = 2
_BM = 128
_CH = 8
_WREP = 128


def _routing_body(nb, lg_ref, bexp_ref, pos_ref, w_ref):
    T, E = lg_ref.shape
    lg = lg_ref[...]
    m = jnp.max(lg, axis=1, keepdims=True)
    ex = jnp.exp(lg - m)
    p = ex / jnp.sum(ex, axis=1, keepdims=True)
    lane = jax.lax.broadcasted_iota(jnp.int32, p.shape, 1)

    w1v = jnp.max(p, axis=1, keepdims=True)
    i1 = jnp.min(jnp.where(p == w1v, lane, E), axis=1, keepdims=True)
    pm = jnp.where(lane == i1, -1.0, p)
    w2v = jnp.max(pm, axis=1, keepdims=True)
    i2 = jnp.min(jnp.where(pm == w2v, lane, E), axis=1, keepdims=True)

    O0 = (lane == i1).astype(jnp.float32)
    O1 = (lane == i2).astype(jnp.float32)
    O = jnp.concatenate([O0, O1], axis=0)

    counts = jnp.sum(O, axis=0, keepdims=True).astype(jnp.int32)
    pcount = ((counts + (_BM - 1)) // _BM) * _BM
    v = pcount
    sh = 1
    while sh < E:
        v = v + jnp.concatenate(
            [jnp.zeros((1, sh), jnp.int32), v[:, :E - sh]], axis=1)
        sh *= 2
    pe_end = v
    po = (pe_end - pcount).astype(jnp.float32)

    nc = (2 * T) // _BM
    sub = jax.lax.broadcasted_iota(jnp.int32, (_BM, _BM), 0)
    lan2 = jax.lax.broadcasted_iota(jnp.int32, (_BM, _BM), 1)
    ls = (lan2 < sub).astype(jnp.float32)
    carry = jnp.zeros((1, E), jnp.float32)
    for c in range(nc):
        oc = O[c * _BM:(c + 1) * _BM]
        rk = jnp.dot(ls, oc, preferred_element_type=jnp.float32) + carry
        pos_c = jnp.sum((rk + po) * oc, axis=1, keepdims=True)
        pos_ref[c * _BM:(c + 1) * _BM, :] = pos_c.astype(jnp.int32)
        carry = carry + jnp.sum(oc, axis=0, keepdims=True)

    wcat = jnp.concatenate([w1v, w2v], axis=0)
    w_ref[...] = wcat * jnp.ones((1, _WREP), jnp.float32)

    bv = jax.lax.broadcasted_iota(jnp.int32, (nb, 1), 0) * _BM
    be = jnp.sum((bv >= pe_end).astype(jnp.int32), axis=1, keepdims=True)
    bexp_ref[...] = jnp.minimum(be, E - 1)


def _mlp_body(bexp_sref, ws_ref, xs_ref, W1_ref, b1_ref, W2_ref, b2_ref,
              y_ref):
    h = jnp.dot(xs_ref[...].astype(jnp.bfloat16), W1_ref[0],
                preferred_element_type=jnp.float32) + b1_ref[0]
    h = jax.nn.gelu(h).astype(jnp.bfloat16)
    y = jnp.dot(h, W2_ref[0], preferred_element_type=jnp.float32) + b2_ref[0]
    y_ref[...] = y * ws_ref[:, :1]


def _sc_meshes():
    info = plsc.get_sparse_core_info()
    mesh = plsc.VectorSubcoreMesh(core_axis_name="c", subcore_axis_name="s")
    return info, mesh


def _dispatch_sc(x, idx, wrep, R):
    T, D = x.shape
    J = idx.shape[0]
    info, mesh = _sc_meshes()
    nw = info.num_cores * info.num_subcores
    per_w = J // nw

    @functools.partial(
        pl.kernel,
        out_type=[
            jax.ShapeDtypeStruct((R, D), x.dtype),
            jax.ShapeDtypeStruct((R, _WREP), jnp.float32),
        ],
        mesh=mesh,
        scratch_types=[
            pltpu.VMEM((_CH,), jnp.int32),
            pltpu.VMEM((_CH, D), x.dtype),
            pltpu.VMEM((_CH, _WREP), jnp.float32),
        ],
    )
    def k(x_hbm, idx_hbm, w_hbm, xs_hbm, ws_hbm, idx_v, rows_v, w_v):
        wid = lax.axis_index("s") * info.num_cores + lax.axis_index("c")
        base = wid * per_w

        @pl.loop(0, per_w, step=_CH)
        def _(c):
            b = base + c
            sb = lax.rem(b, T)
            pltpu.sync_copy(idx_hbm.at[pl.ds(b, _CH)], idx_v)
            pltpu.sync_copy(x_hbm.at[pl.ds(sb, _CH)], rows_v)
            pltpu.sync_copy(w_hbm.at[pl.ds(b, _CH)], w_v)
            pltpu.sync_copy(rows_v, xs_hbm.at[idx_v])
            pltpu.sync_copy(w_v, ws_hbm.at[idx_v])

    return k(x, idx, wrep)


def _combine_sc(y, i0, i1):
    R, D = y.shape
    T = i0.shape[0]
    info, mesh = _sc_meshes()
    nw = info.num_cores * info.num_subcores
    nl = info.num_lanes
    per_w = T // nw

    @functools.partial(
        pl.kernel,
        out_type=jax.ShapeDtypeStruct((T, D), jnp.float32),
        mesh=mesh,
        scratch_types=[
            pltpu.VMEM((_CH,), jnp.int32),
            pltpu.VMEM((_CH,), jnp.int32),
            pltpu.VMEM((_CH, D), jnp.float32),
            pltpu.VMEM((_CH, D), jnp.float32),
            pltpu.VMEM((_CH, D), jnp.float32),
            pltpu.SemaphoreType.DMA,
            pltpu.SemaphoreType.DMA,
        ],
    )
    def k(y_hbm, i0_hbm, i1_hbm, o_hbm, i0_v, i1_v, g0_v, g1_v, o_v, s0, s1):
        wid = lax.axis_index("s") * info.num_cores + lax.axis_index("c")
        base = wid * per_w

        @pl.loop(0, per_w, step=_CH)
        def _(c):
            b = base + c
            pltpu.sync_copy(i0_hbm.at[pl.ds(b, _CH)], i0_v)
            pltpu.sync_copy(i1_hbm.at[pl.ds(b, _CH)], i1_v)
            cp0 = pltpu.async_copy(y_hbm.at[i0_v], g0_v, s0)
            cp1 = pltpu.async_copy(y_hbm.at[i1_v], g1_v, s1)
            cp0.wait()
            cp1.wait()
            for r in range(_CH):
                @pl.loop(0, D, step=nl)
                def _(c16):
                    o_v[r, pl.ds(c16, nl)] = (
                        g0_v[r, pl.ds(c16, nl)] + g1_v[r, pl.ds(c16, nl)])
            pltpu.sync_copy(o_v, o_hbm.at[pl.ds(b, _CH)])

    return k(y, i0, i1)


@jax.jit
def kernel(hidden_states, router_logits, W1, b1, W2, b2):
    B_, S_, D_ = hidden_states.shape
    T = B_ * S_
    E_, _, F_ = W1.shape
    NB = (T * _K + E_ * (_BM - 1) + _BM - 1) // _BM
    R = NB * _BM

    x = hidden_states.reshape(T, D_)
    W1b = W1.astype(jnp.bfloat16)
    W2b = W2.astype(jnp.bfloat16)
    b1r = b1.reshape(E_, 1, F_)
    b2r = b2.reshape(E_, 1, D_)

    bexp, pos, wrep = pl.pallas_call(
        lambda *refs: _routing_body(NB, *refs),
        out_shape=[
            jax.ShapeDtypeStruct((NB, 1), jnp.int32),
            jax.ShapeDtypeStruct((2 * T, 1), jnp.int32),
            jax.ShapeDtypeStruct((2 * T, _WREP), jnp.float32),
        ],
    )(router_logits)

    idx = pos.reshape(2 * T)
    xs, ws = _dispatch_sc(x, idx, wrep, R)

    y = pl.pallas_call(
        _mlp_body,
        grid_spec=pltpu.PrefetchScalarGridSpec(
            num_scalar_prefetch=1,
            grid=(NB,),
            in_specs=[
                pl.BlockSpec((_BM, _WREP), lambda b, be: (b, 0)),
                pl.BlockSpec((_BM, D_), lambda b, be: (b, 0)),
                pl.BlockSpec((1, D_, F_), lambda b, be: (be[b], 0, 0)),
                pl.BlockSpec((1, 1, F_), lambda b, be: (be[b], 0, 0)),
                pl.BlockSpec((1, F_, D_), lambda b, be: (be[b], 0, 0)),
                pl.BlockSpec((1, 1, D_), lambda b, be: (be[b], 0, 0)),
            ],
            out_specs=pl.BlockSpec((_BM, D_), lambda b, be: (b, 0)),
        ),
        out_shape=jax.ShapeDtypeStruct((R, D_), jnp.float32),
        compiler_params=pltpu.CompilerParams(
            dimension_semantics=("arbitrary",)),
    )(bexp.reshape(NB), ws, xs, W1b, b1r, W2b, b2r)

    out = _combine_sc(y, idx[:T], idx[T:])
    return out.reshape(B_, S_, D_)


# speedup vs baseline: 1.0410x; 1.0410x over previous
import functools

import jax
import jax.numpy as jnp
from jax import lax
from jax.experimental import pallas as pl
from jax.experimental.pallas import tpu as pltpu
from jax.experimental.pallas import tpu_sc as plsc

_K = 2
_BM = 128
_BMP = 256
_CH = 8
_WREP = 128


def _routing_body(nb, lg_ref, bexp_ref, pos_ref, w_ref):
    T, E = lg_ref.shape
    lg = lg_ref[...]
    m = jnp.max(lg, axis=1, keepdims=True)
    ex = jnp.exp(lg - m)
    p = ex / jnp.sum(ex, axis=1, keepdims=True)
    lane = jax.lax.broadcasted_iota(jnp.int32, p.shape, 1)

    w1v = jnp.max(p, axis=1, keepdims=True)
    i1 = jnp.min(jnp.where(p == w1v, lane, E), axis=1, keepdims=True)
    pm = jnp.where(lane == i1, -1.0, p)
    w2v = jnp.max(pm, axis=1, keepdims=True)
    i2 = jnp.min(jnp.where(pm == w2v, lane, E), axis=1, keepdims=True)

    O0 = (lane == i1).astype(jnp.float32)
    O1 = (lane == i2).astype(jnp.float32)
    O = jnp.concatenate([O0, O1], axis=0)

    counts = jnp.sum(O, axis=0, keepdims=True).astype(jnp.int32)
    pcount = ((counts + (_BMP - 1)) // _BMP) * _BMP
    v = pcount
    sh = 1
    while sh < E:
        v = v + jnp.concatenate(
            [jnp.zeros((1, sh), jnp.int32), v[:, :E - sh]], axis=1)
        sh *= 2
    pe_end = v
    po = (pe_end - pcount).astype(jnp.float32)

    nc = (2 * T) // _BM
    sub = jax.lax.broadcasted_iota(jnp.int32, (_BM, _BM), 0)
    lan2 = jax.lax.broadcasted_iota(jnp.int32, (_BM, _BM), 1)
    ls = (lan2 < sub).astype(jnp.float32)
    carry = jnp.zeros((1, E), jnp.float32)
    for c in range(nc):
        oc = O[c * _BM:(c + 1) * _BM]
        rk = jnp.dot(ls, oc, preferred_element_type=jnp.float32) + carry
        pos_c = jnp.sum((rk + po) * oc, axis=1, keepdims=True)
        pos_ref[c * _BM:(c + 1) * _BM, :] = pos_c.astype(jnp.int32)
        carry = carry + jnp.sum(oc, axis=0, keepdims=True)

    wcat = jnp.concatenate([w1v, w2v], axis=0)
    w_ref[...] = wcat * jnp.ones((1, _WREP), jnp.float32)

    bv = jax.lax.broadcasted_iota(jnp.int32, (nb, 1), 0) * _BMP
    be = jnp.sum((bv >= pe_end).astype(jnp.int32), axis=1, keepdims=True)
    bexp_ref[...] = jnp.minimum(be, E - 1)


def _mlp_body(bexp_sref, ws_ref, xs_ref, W1_ref, b1_ref, W2_ref, b2_ref,
              y_ref):
    h = jnp.dot(xs_ref[...].astype(jnp.bfloat16), W1_ref[0],
                preferred_element_type=jnp.float32) + b1_ref[0]
    h = jax.nn.gelu(h).astype(jnp.bfloat16)
    y = jnp.dot(h, W2_ref[0], preferred_element_type=jnp.float32) + b2_ref[0]
    y_ref[...] = y * ws_ref[:, :1]


def _sc_meshes():
    info = plsc.get_sparse_core_info()
    mesh = plsc.VectorSubcoreMesh(core_axis_name="c", subcore_axis_name="s")
    return info, mesh


def _dispatch_sc(x, idx, wrep, R):
    T, D = x.shape
    J = idx.shape[0]
    info, mesh = _sc_meshes()
    nw = info.num_cores * info.num_subcores
    per_w = J // nw

    @functools.partial(
        pl.kernel,
        out_type=[
            jax.ShapeDtypeStruct((R, D), x.dtype),
            jax.ShapeDtypeStruct((R, _WREP), jnp.float32),
        ],
        mesh=mesh,
        scratch_types=[
            pltpu.VMEM((_CH,), jnp.int32),
            pltpu.VMEM((_CH, D), x.dtype),
            pltpu.VMEM((_CH, _WREP), jnp.float32),
        ],
    )
    def k(x_hbm, idx_hbm, w_hbm, xs_hbm, ws_hbm, idx_v, rows_v, w_v):
        wid = lax.axis_index("s") * info.num_cores + lax.axis_index("c")
        base = wid * per_w

        @pl.loop(0, per_w, step=_CH)
        def _(c):
            b = base + c
            sb = lax.rem(b, T)
            pltpu.sync_copy(idx_hbm.at[pl.ds(b, _CH)], idx_v)
            pltpu.sync_copy(x_hbm.at[pl.ds(sb, _CH)], rows_v)
            pltpu.sync_copy(w_hbm.at[pl.ds(b, _CH)], w_v)
            pltpu.sync_copy(rows_v, xs_hbm.at[idx_v])
            pltpu.sync_copy(w_v, ws_hbm.at[idx_v])

    return k(x, idx, wrep)


def _combine_sc(y, i0, i1):
    R, D = y.shape
    T = i0.shape[0]
    info, mesh = _sc_meshes()
    nw = info.num_cores * info.num_subcores
    nl = info.num_lanes
    per_w = T // nw

    @functools.partial(
        pl.kernel,
        out_type=jax.ShapeDtypeStruct((T, D), jnp.float32),
        mesh=mesh,
        scratch_types=[
            pltpu.VMEM((_CH,), jnp.int32),
            pltpu.VMEM((_CH,), jnp.int32),
            pltpu.VMEM((_CH, D), jnp.float32),
            pltpu.VMEM((_CH, D), jnp.float32),
            pltpu.VMEM((_CH, D), jnp.float32),
            pltpu.SemaphoreType.DMA,
            pltpu.SemaphoreType.DMA,
        ],
    )
    def k(y_hbm, i0_hbm, i1_hbm, o_hbm, i0_v, i1_v, g0_v, g1_v, o_v, s0, s1):
        wid = lax.axis_index("s") * info.num_cores + lax.axis_index("c")
        base = wid * per_w

        @pl.loop(0, per_w, step=_CH)
        def _(c):
            b = base + c
            pltpu.sync_copy(i0_hbm.at[pl.ds(b, _CH)], i0_v)
            pltpu.sync_copy(i1_hbm.at[pl.ds(b, _CH)], i1_v)
            cp0 = pltpu.async_copy(y_hbm.at[i0_v], g0_v, s0)
            cp1 = pltpu.async_copy(y_hbm.at[i1_v], g1_v, s1)
            cp0.wait()
            cp1.wait()
            for r in range(_CH):
                @pl.loop(0, D, step=nl)
                def _(c16):
                    o_v[r, pl.ds(c16, nl)] = (
                        g0_v[r, pl.ds(c16, nl)] + g1_v[r, pl.ds(c16, nl)])
            pltpu.sync_copy(o_v, o_hbm.at[pl.ds(b, _CH)])

    return k(y, i0, i1)


@jax.jit
def kernel(hidden_states, router_logits, W1, b1, W2, b2):
    B_, S_, D_ = hidden_states.shape
    T = B_ * S_
    E_, _, F_ = W1.shape
    NB = (T * _K + E_ * (_BMP - 1) + _BMP - 1) // _BMP
    R = NB * _BMP

    x = hidden_states.reshape(T, D_)
    W1b = W1.astype(jnp.bfloat16)
    W2b = W2.astype(jnp.bfloat16)
    b1r = b1.reshape(E_, 1, F_)
    b2r = b2.reshape(E_, 1, D_)

    bexp, pos, wrep = pl.pallas_call(
        lambda *refs: _routing_body(NB, *refs),
        out_shape=[
            jax.ShapeDtypeStruct((NB, 1), jnp.int32),
            jax.ShapeDtypeStruct((2 * T, 1), jnp.int32),
            jax.ShapeDtypeStruct((2 * T, _WREP), jnp.float32),
        ],
    )(router_logits)

    idx = pos.reshape(2 * T)
    xs, ws = _dispatch_sc(x, idx, wrep, R)

    y = pl.pallas_call(
        _mlp_body,
        grid_spec=pltpu.PrefetchScalarGridSpec(
            num_scalar_prefetch=1,
            grid=(NB,),
            in_specs=[
                pl.BlockSpec((_BMP, _WREP), lambda b, be: (b, 0)),
                pl.BlockSpec((_BMP, D_), lambda b, be: (b, 0)),
                pl.BlockSpec((1, D_, F_), lambda b, be: (be[b], 0, 0)),
                pl.BlockSpec((1, 1, F_), lambda b, be: (be[b], 0, 0)),
                pl.BlockSpec((1, F_, D_), lambda b, be: (be[b], 0, 0)),
                pl.BlockSpec((1, 1, D_), lambda b, be: (be[b], 0, 0)),
            ],
            out_specs=pl.BlockSpec((_BMP, D_), lambda b, be: (b, 0)),
        ),
        out_shape=jax.ShapeDtypeStruct((R, D_), jnp.float32),
        compiler_params=pltpu.CompilerParams(
            dimension_semantics=("arbitrary",)),
    )(bexp.reshape(NB), ws, xs, W1b, b1r, W2b, b2r)

    out = _combine_sc(y, idx[:T], idx[T:])
    return out.reshape(B_, S_, D_)
